# baseline (speedup 1.0000x reference)
import functools
import math

import jax
import jax.numpy as jnp
from jax.experimental import pallas as pl
from jax.experimental.pallas import tpu as pltpu

ALPHA = 1e-4
BETA = 1.0
MAX_STD = 1000.0
MIN_STD = 1.0

_INV_SQRT_2PI = 1.0 / math.sqrt(2.0 * math.pi)

_N_STEPS = 8
_ROWS_PER_STEP = 1024


def _fused_body(logits_ref, labels_ref, aw_ref, len_ref, params_ref,
                loss_ref, nll_ref, acc_ref):
    i = pl.program_id(0)

    part = jnp.sum(jnp.abs(params_ref[...]))

    @pl.when(i == 0)
    def _first():
        logits = logits_ref[...]
        m = jnp.max(logits, axis=1, keepdims=True)
        lse = m + jnp.log(jnp.sum(jnp.exp(logits - m), axis=1, keepdims=True))
        logp = logits - lse
        labels = labels_ref[...]
        picked = jnp.where(labels == 1, logp[:, 1:2], logp[:, 0:1])
        nll = -jnp.mean(picked)

        aw = aw_ref[...]
        b, seg_len = aw.shape
        idx = jax.lax.broadcasted_iota(jnp.int32, (b, seg_len), 1)
        x = (idx.astype(jnp.float32) + 1.0) / seg_len
        sums = jnp.sum(aw, axis=1, keepdims=True)
        means = jnp.sum(x * aw, axis=1, keepdims=True) / sums
        len_f = len_ref[...].astype(jnp.float32)
        ideal_stds = jnp.where(labels == 1, MIN_STD / len_f, MAX_STD / len_f)
        z = (x - means) / ideal_stds
        r_hats = jnp.exp(-0.5 * z * z) * (_INV_SQRT_2PI / ideal_stds)
        rs = r_hats / (jnp.sum(r_hats, axis=1, keepdims=True) + 1e-6)
        diff = aw - rs
        aw_penalty = (BETA / 2.0) * jnp.mean(diff * diff)

        nll_ref[...] = jnp.reshape(nll, (1, 1))
        acc_ref[0] = part
        acc_ref[1] = nll + aw_penalty

    @pl.when(i > 0)
    def _rest():
        acc_ref[0] += part

    @pl.when(i == _N_STEPS - 1)
    def _last():
        loss_ref[...] = jnp.reshape(
            acc_ref[1] + (ALPHA / 2.0) * acc_ref[0], (1, 1))


@jax.jit
def _run(logits, labels2d, aw2d, lengths2d, params2d):
    b, seg_len = aw2d.shape
    zero2 = lambda i: (0, 0)
    out = pl.pallas_call(
        _fused_body,
        grid=(_N_STEPS,),
        in_specs=[
            pl.BlockSpec((b, 2), zero2),
            pl.BlockSpec((b, 1), zero2),
            pl.BlockSpec((b, seg_len), zero2),
            pl.BlockSpec((b, 1), zero2),
            pl.BlockSpec((_ROWS_PER_STEP, 128), lambda i: (i, 0)),
        ],
        out_specs=(
            pl.BlockSpec((1, 1), zero2),
            pl.BlockSpec((1, 1), zero2),
        ),
        out_shape=(
            jax.ShapeDtypeStruct((1, 1), jnp.float32),
            jax.ShapeDtypeStruct((1, 1), jnp.float32),
        ),
        scratch_shapes=[pltpu.SMEM((2,), jnp.float32)],
    )(logits, labels2d, aw2d, lengths2d, params2d)
    return out


def kernel(logits, labels, attention_weights, lengths, params):
    b = lengths.shape[0]
    seg_len = attention_weights.shape[0] // b
    aw2d = attention_weights.reshape(b, seg_len)
    labels2d = labels.astype(jnp.int32).reshape(b, 1)
    lengths2d = lengths.reshape(b, 1)
    params2d = params.reshape(-1, 128)
    loss, nll = _run(logits, labels2d, aw2d, lengths2d, params2d)
    return (loss[0, 0], nll[0, 0])


# speedup vs baseline: 1.0008x; 1.0008x over previous
import functools
import math

import jax
import jax.numpy as jnp
from jax.experimental import pallas as pl
from jax.experimental.pallas import tpu as pltpu

ALPHA = 1e-4
BETA = 1.0
MAX_STD = 1000.0
MIN_STD = 1.0

_INV_SQRT_2PI = 1.0 / math.sqrt(2.0 * math.pi)

_N_CHUNKS = 8
_ROWS_PER_CHUNK = 1024


def _fused_body(logits_ref, labels_ref, aw_ref, len_ref, params_hbm,
                loss_ref, nll_ref, buf0, buf1, sem0, sem1):
    bufs = (buf0, buf1)
    sems = (sem0, sem1)

    def start(c):
        pltpu.make_async_copy(
            params_hbm.at[pl.ds(c * _ROWS_PER_CHUNK, _ROWS_PER_CHUNK), :],
            bufs[c % 2], sems[c % 2]).start()

    start(0)
    start(1)

    logits = logits_ref[...]
    m = jnp.max(logits, axis=1, keepdims=True)
    lse = m + jnp.log(jnp.sum(jnp.exp(logits - m), axis=1, keepdims=True))
    logp = logits - lse
    labels = labels_ref[...]
    picked = jnp.where(labels == 1, logp[:, 1:2], logp[:, 0:1])
    nll = -jnp.mean(picked)

    aw = aw_ref[...]
    b, seg_len = aw.shape
    idx = jax.lax.broadcasted_iota(jnp.int32, (b, seg_len), 1)
    x = (idx.astype(jnp.float32) + 1.0) / seg_len
    sums = jnp.sum(aw, axis=1, keepdims=True)
    means = jnp.sum(x * aw, axis=1, keepdims=True) / sums
    len_f = len_ref[...].astype(jnp.float32)
    ideal_stds = jnp.where(labels == 1, MIN_STD / len_f, MAX_STD / len_f)
    z = (x - means) / ideal_stds
    r_hats = jnp.exp(-0.5 * z * z) * (_INV_SQRT_2PI / ideal_stds)
    rs = r_hats / (jnp.sum(r_hats, axis=1, keepdims=True) + 1e-6)
    diff = aw - rs
    aw_penalty = (BETA / 2.0) * jnp.mean(diff * diff)

    acc = jnp.zeros((8, 128), jnp.float32)
    for c in range(_N_CHUNKS):
        pltpu.make_async_copy(
            params_hbm.at[pl.ds(c * _ROWS_PER_CHUNK, _ROWS_PER_CHUNK), :],
            bufs[c % 2], sems[c % 2]).wait()
        p = jnp.abs(bufs[c % 2][...])
        if c + 2 < _N_CHUNKS:
            pass_acc = jnp.sum(p.reshape(128, 8, 128), axis=0)
            start(c + 2)
        else:
            pass_acc = jnp.sum(p.reshape(128, 8, 128), axis=0)
        acc = acc + pass_acc
    penalty = (ALPHA / 2.0) * jnp.sum(acc)

    nll_ref[...] = jnp.reshape(nll, (1, 1))
    loss_ref[...] = jnp.reshape(nll + aw_penalty + penalty, (1, 1))


@jax.jit
def _run(logits, labels2d, aw2d, lengths2d, params2d):
    b, seg_len = aw2d.shape
    out = pl.pallas_call(
        _fused_body,
        in_specs=[
            pl.BlockSpec(memory_space=pltpu.VMEM),
            pl.BlockSpec(memory_space=pltpu.VMEM),
            pl.BlockSpec(memory_space=pltpu.VMEM),
            pl.BlockSpec(memory_space=pltpu.VMEM),
            pl.BlockSpec(memory_space=pltpu.HBM),
        ],
        out_specs=(
            pl.BlockSpec(memory_space=pltpu.VMEM),
            pl.BlockSpec(memory_space=pltpu.VMEM),
        ),
        out_shape=(
            jax.ShapeDtypeStruct((1, 1), jnp.float32),
            jax.ShapeDtypeStruct((1, 1), jnp.float32),
        ),
        scratch_shapes=[
            pltpu.VMEM((_ROWS_PER_CHUNK, 128), jnp.float32),
            pltpu.VMEM((_ROWS_PER_CHUNK, 128), jnp.float32),
            pltpu.SemaphoreType.DMA,
            pltpu.SemaphoreType.DMA,
        ],
    )(logits, labels2d, aw2d, lengths2d, params2d)
    return out


def kernel(logits, labels, attention_weights, lengths, params):
    b = lengths.shape[0]
    seg_len = attention_weights.shape[0] // b
    aw2d = attention_weights.reshape(b, seg_len)
    labels2d = labels.astype(jnp.int32).reshape(b, 1)
    lengths2d = lengths.reshape(b, 1)
    params2d = params.reshape(-1, 128)
    loss, nll = _run(logits, labels2d, aw2d, lengths2d, params2d)
    return (loss[0, 0], nll[0, 0])


# speedup vs baseline: 1.9735x; 1.9719x over previous
import math

import jax
import jax.numpy as jnp
from jax.experimental import pallas as pl
from jax.experimental.pallas import tpu as pltpu

ALPHA = 1e-4
BETA = 1.0
MAX_STD = 1000.0
MIN_STD = 1.0

_INV_SQRT_2PI = 1.0 / math.sqrt(2.0 * math.pi)


def _fused_body(logits_ref, labels_ref, aw_ref, len_ref, params_ref,
                loss_ref, nll_ref):
    logits = logits_ref[...]
    b = logits.shape[0]
    m = jnp.max(logits, axis=1, keepdims=True)
    lse = m + jnp.log(jnp.sum(jnp.exp(logits - m), axis=1, keepdims=True))
    logp = logits - lse
    labels = labels_ref[...].reshape(b, 1)
    picked = jnp.where(labels == 1, logp[:, 1:2], logp[:, 0:1])
    nll = -jnp.mean(picked)

    aw = aw_ref[...].reshape(b, -1)
    seg_len = aw.shape[1]
    idx = jax.lax.broadcasted_iota(jnp.int32, (b, seg_len), 1)
    x = (idx.astype(jnp.float32) + 1.0) / seg_len
    sums = jnp.sum(aw, axis=1, keepdims=True)
    means = jnp.sum(x * aw, axis=1, keepdims=True) / sums
    len_f = len_ref[...].reshape(b, 1).astype(jnp.float32)
    ideal_stds = jnp.where(labels == 1, MIN_STD / len_f, MAX_STD / len_f)
    z = (x - means) / ideal_stds
    r_hats = jnp.exp(-0.5 * z * z) * (_INV_SQRT_2PI / ideal_stds)
    rs = r_hats / (jnp.sum(r_hats, axis=1, keepdims=True) + 1e-6)
    diff = aw - rs
    aw_penalty = (BETA / 2.0) * jnp.mean(diff * diff)

    p = params_ref[...].reshape(-1, 512)
    penalty = (ALPHA / 2.0) * jnp.sum(jnp.abs(p))

    nll_ref[0] = nll
    loss_ref[0] = nll + penalty + aw_penalty


@jax.jit
def _run(logits, labels, attention_weights, lengths, params):
    vmem = pl.BlockSpec(memory_space=pltpu.VMEM)
    smem = pl.BlockSpec(memory_space=pltpu.SMEM)
    out = pl.pallas_call(
        _fused_body,
        in_specs=[vmem] * 5,
        out_specs=(smem, smem),
        out_shape=(
            jax.ShapeDtypeStruct((1,), jnp.float32),
            jax.ShapeDtypeStruct((1,), jnp.float32),
        ),
    )(logits, labels, attention_weights, lengths, params)
    return out[0][0], out[1][0]


def kernel(logits, labels, attention_weights, lengths, params):
    return _run(logits, labels, attention_weights, lengths, params)


# speedup vs baseline: 2.0126x; 1.0198x over previous
import math

import jax
import jax.numpy as jnp
from jax.experimental import pallas as pl
from jax.experimental.pallas import tpu as pltpu

ALPHA = 1e-4
BETA = 1.0
MAX_STD = 1000.0
MIN_STD = 1.0

_INV_SQRT_2PI = 1.0 / math.sqrt(2.0 * math.pi)


def _fused_body(logits_ref, labels_ref, aw_ref, len_ref, params_ref,
                loss_ref, nll_ref):
    logits = logits_ref[...]
    b = logits.shape[0]
    m = jnp.max(logits, axis=1, keepdims=True)
    lse = m + jnp.log(jnp.sum(jnp.exp(logits - m), axis=1, keepdims=True))
    logp = logits - lse
    labels = labels_ref[...].reshape(b, 1)
    picked = jnp.where(labels == 1, logp[:, 1:2], logp[:, 0:1])
    nll = -jnp.mean(picked)

    aw = aw_ref[...].reshape(b, -1)
    seg_len = aw.shape[1]
    idx = jax.lax.broadcasted_iota(jnp.int32, (b, seg_len), 1)
    x = (idx.astype(jnp.float32) + 1.0) / seg_len
    sums = jnp.sum(aw, axis=1, keepdims=True)
    means = jnp.sum(x * aw, axis=1, keepdims=True) / sums
    len_f = len_ref[...].reshape(b, 1).astype(jnp.float32)
    ideal_stds = jnp.where(labels == 1, MIN_STD / len_f, MAX_STD / len_f)
    z = (x - means) / ideal_stds
    r_hats = jnp.exp(-0.5 * z * z) * (_INV_SQRT_2PI / ideal_stds)
    rs = r_hats / (jnp.sum(r_hats, axis=1, keepdims=True) + 1e-6)
    diff = aw - rs
    aw_penalty = (BETA / 2.0) * jnp.mean(diff * diff)

    p = params_ref[...].reshape(-1, 512)
    penalty = (ALPHA / 2.0) * jnp.sum(jnp.abs(p))

    nll_ref[...] = nll
    loss_ref[...] = nll + penalty + aw_penalty


@jax.jit
def _run(logits, labels, attention_weights, lengths, params):
    vmem = pl.BlockSpec(memory_space=pltpu.VMEM)
    smem = pl.BlockSpec(memory_space=pltpu.SMEM)
    out = pl.pallas_call(
        _fused_body,
        in_specs=[vmem] * 5,
        out_specs=(smem, smem),
        out_shape=(
            jax.ShapeDtypeStruct((), jnp.float32),
            jax.ShapeDtypeStruct((), jnp.float32),
        ),
    )(logits, labels, attention_weights, lengths, params)
    return out


def kernel(logits, labels, attention_weights, lengths, params):
    return _run(logits, labels, attention_weights, lengths, params)


# speedup vs baseline: 2.4754x; 1.2300x over previous
import math

import jax
import jax.numpy as jnp
from jax.experimental import pallas as pl
from jax.experimental.pallas import tpu as pltpu

ALPHA = 1e-4
BETA = 1.0
MAX_STD = 1000.0
MIN_STD = 1.0

_INV_SQRT_2PI = 1.0 / math.sqrt(2.0 * math.pi)


def _fused_body(logits_ref, labels_ref, aw_ref, len_ref, params_ref,
                loss_ref, nll_ref):
    lt = logits_ref[...]
    b = lt.shape[1]
    l0 = lt[0:1, :]
    l1 = lt[1:2, :]
    m = jnp.maximum(l0, l1)
    lse = m + jnp.log(jnp.exp(l0 - m) + jnp.exp(l1 - m))
    lab_row = labels_ref[...].reshape(1, b)
    picked = jnp.where(lab_row == 1, l1, l0) - lse
    nll = -jnp.sum(picked) / b
    labels = labels_ref[...].reshape(b, 1)

    aw = aw_ref[...].reshape(b, -1)
    seg_len = aw.shape[1]
    idx = jax.lax.broadcasted_iota(jnp.int32, (b, seg_len), 1)
    x = (idx.astype(jnp.float32) + 1.0) / seg_len
    sums = jnp.sum(aw, axis=1, keepdims=True)
    means = jnp.sum(x * aw, axis=1, keepdims=True) / sums
    len_f = len_ref[...].reshape(b, 1).astype(jnp.float32)
    ideal_stds = jnp.where(labels == 1, MIN_STD / len_f, MAX_STD / len_f)
    z = (x - means) / ideal_stds
    r_hats = jnp.exp(-0.5 * z * z) * (_INV_SQRT_2PI / ideal_stds)
    rs = r_hats / (jnp.sum(r_hats, axis=1, keepdims=True) + 1e-6)
    diff = aw - rs
    aw_penalty = (BETA / 2.0) * jnp.mean(diff * diff)

    p = params_ref[...].reshape(-1, 512)
    penalty = (ALPHA / 2.0) * jnp.sum(jnp.abs(p))

    nll_ref[...] = nll
    loss_ref[...] = nll + penalty + aw_penalty


@jax.jit
def _run(logits, labels, attention_weights, lengths, params):
    vmem = pl.BlockSpec(memory_space=pltpu.VMEM)
    smem = pl.BlockSpec(memory_space=pltpu.SMEM)
    out = pl.pallas_call(
        _fused_body,
        in_specs=[vmem] * 5,
        out_specs=(smem, smem),
        out_shape=(
            jax.ShapeDtypeStruct((), jnp.float32),
            jax.ShapeDtypeStruct((), jnp.float32),
        ),
    )(logits.T, labels, attention_weights, lengths, params)
    return out


def kernel(logits, labels, attention_weights, lengths, params):
    return _run(logits, labels, attention_weights, lengths, params)
